# baseline (speedup 1.0000x reference)
import jax
import jax.numpy as jnp
import numpy as np
from jax.experimental import pallas as pl

B, N, S, K, D = 4, 8192, 1024, 32, 64
LEAKY = 0.1


def _fps(xyz, npoint):
    b, n, _ = xyz.shape
    idx0 = jnp.zeros((b, npoint), dtype=jnp.int32)
    dists = jnp.full((b, n), 1e10, dtype=xyz.dtype)
    last = jnp.zeros((b,), dtype=jnp.int32)
    def body(i, state):
        idx, dd, lst = state
        last_pt = jnp.take_along_axis(xyz, lst[:, None, None].astype(jnp.int32), axis=1)
        d = jnp.sum((xyz - last_pt) ** 2, axis=-1)
        dd = jnp.minimum(dd, d)
        nxt = jnp.argmax(dd, axis=1).astype(jnp.int32)
        idx = idx.at[:, i].set(nxt)
        return (idx, dd, nxt)
    idx, _, _ = jax.lax.fori_loop(1, npoint, body, (idx0, dists, last))
    return idx


def _group(points, idx):
    return jax.vmap(lambda p, i: p[i])(points, idx)


def _square_distance(src, dst):
    d = -2.0 * jnp.einsum('bnc,bmc->bnm', src, dst)
    d = d + jnp.sum(src ** 2, -1)[:, :, None]
    d = d + jnp.sum(dst ** 2, -1)[:, None, :]
    return d


def _bn2d(x, g, b):
    m = jnp.mean(x, axis=(0, 2, 3), keepdims=True)
    v = jnp.var(x, axis=(0, 2, 3), keepdims=True)
    return (x - m) / jnp.sqrt(v + 1e-5) * g[None, :, None, None] + b[None, :, None, None]


def _lrelu(x):
    return jnp.where(x >= 0, x, LEAKY * x)


def _identity_pallas(x):
    def body(x_ref, o_ref):
        o_ref[...] = x_ref[...]
    return pl.pallas_call(
        body, out_shape=jax.ShapeDtypeStruct(x.shape, x.dtype))(x)


def kernel(xyz, points, W0, W1, W2, g0, g1, g2, b0, b1, b2):
    xyz_t = jnp.transpose(xyz, (0, 2, 1))
    pts_t = jnp.transpose(points, (0, 2, 1))
    fps_idx = _fps(jax.lax.stop_gradient(xyz_t), S)
    new_xyz = jax.vmap(lambda p, i: p[i])(xyz_t, fps_idx)
    sqr = _square_distance(new_xyz, xyz_t)
    _, knn_idx = jax.lax.top_k(-sqr, K)
    grouped_xyz = _group(xyz_t, knn_idx)
    grouped_norm = grouped_xyz - new_xyz[:, :, None, :]
    grouped_pts = _group(pts_t, knn_idx)
    feat = jnp.concatenate([grouped_norm, grouped_pts], axis=-1)
    x = jnp.transpose(feat, (0, 3, 1, 2))
    x = _lrelu(_bn2d(jnp.einsum('bcsk,cd->bdsk', x, W0), g0, b0))
    x = _lrelu(_bn2d(jnp.einsum('bcsk,cd->bdsk', x, W1), g1, b1))
    x = _lrelu(_bn2d(jnp.einsum('bcsk,cd->bdsk', x, W2), g2, b2))
    x = jnp.max(x, axis=-1)
    x = _identity_pallas(x)
    return (jnp.transpose(new_xyz, (0, 2, 1)), x, fps_idx)


# speedup vs baseline: 1.7539x; 1.7539x over previous
import jax
import jax.numpy as jnp
import numpy as np
from jax.experimental import pallas as pl
from jax.experimental.pallas import tpu as pltpu

B, N, S, K, D = 4, 8192, 1024, 32, 64
LEAKY = 0.1


def _fps_body(xyz_ref, idx_ref, nxyz_ref, dd_ref):
    x = xyz_ref[:, 0, :]
    y = xyz_ref[:, 1, :]
    z = xyz_ref[:, 2, :]
    iota = jax.lax.broadcasted_iota(jnp.int32, (B, N), 1)
    lane = jax.lax.broadcasted_iota(jnp.int32, (B, 128), 1)

    dd_ref[...] = jnp.full((B, N), 1e10, jnp.float32)

    def body(j, carry):
        cx, cy, cz, ai, ax, ay, az = carry
        dx = x - cx
        dy = y - cy
        dz = z - cz
        d = dx * dx + dy * dy + dz * dz
        dd = jnp.minimum(dd_ref[...], d)
        dd_ref[...] = dd
        m = jnp.max(dd, axis=1, keepdims=True)
        cand = jnp.where(dd == m, iota, N)
        nxt = jnp.min(cand, axis=1, keepdims=True)
        msk = iota == nxt
        ncx = jnp.sum(jnp.where(msk, x, 0.0), axis=1, keepdims=True)
        ncy = jnp.sum(jnp.where(msk, y, 0.0), axis=1, keepdims=True)
        ncz = jnp.sum(jnp.where(msk, z, 0.0), axis=1, keepdims=True)
        sel = lane == j
        ai = jnp.where(sel, nxt, ai)
        ax = jnp.where(sel, ncx, ax)
        ay = jnp.where(sel, ncy, ay)
        az = jnp.where(sel, ncz, az)
        return (ncx, ncy, ncz, ai, ax, ay, az)

    zi = jnp.zeros((B, 128), jnp.int32)
    zf = jnp.zeros((B, 128), jnp.float32)
    carry = (x[:, 0:1], y[:, 0:1], z[:, 0:1], zi,
             jnp.where(lane == 0, x[:, 0:1], zf),
             jnp.where(lane == 0, y[:, 0:1], zf),
             jnp.where(lane == 0, z[:, 0:1], zf))
    for chunk in range(S // 128):
        lo = 1 if chunk == 0 else 0
        carry = jax.lax.fori_loop(lo, 128, body, carry)
        cx, cy, cz, ai, ax, ay, az = carry
        sl = slice(chunk * 128, (chunk + 1) * 128)
        idx_ref[:, sl] = ai
        nxyz_ref[:, 0, sl] = ax
        nxyz_ref[:, 1, sl] = ay
        nxyz_ref[:, 2, sl] = az
        carry = (cx, cy, cz, zi, zf, zf, zf)


def _fps_pallas(xyz):
    return pl.pallas_call(
        _fps_body,
        out_shape=(
            jax.ShapeDtypeStruct((B, S), jnp.int32),
            jax.ShapeDtypeStruct((B, 3, S), jnp.float32),
        ),
        scratch_shapes=[pltpu.VMEM((B, N), jnp.float32)],
    )(xyz)


def _group(points, idx):
    return jax.vmap(lambda p, i: p[i])(points, idx)


def _square_distance(src, dst):
    d = -2.0 * jnp.einsum('bnc,bmc->bnm', src, dst)
    d = d + jnp.sum(src ** 2, -1)[:, :, None]
    d = d + jnp.sum(dst ** 2, -1)[:, None, :]
    return d


def _bn2d(x, g, b):
    m = jnp.mean(x, axis=(0, 2, 3), keepdims=True)
    v = jnp.var(x, axis=(0, 2, 3), keepdims=True)
    return (x - m) / jnp.sqrt(v + 1e-5) * g[None, :, None, None] + b[None, :, None, None]


def _lrelu(x):
    return jnp.where(x >= 0, x, LEAKY * x)


def kernel(xyz, points, W0, W1, W2, g0, g1, g2, b0, b1, b2):
    xyz_t = jnp.transpose(xyz, (0, 2, 1))
    pts_t = jnp.transpose(points, (0, 2, 1))
    fps_idx, new_xyz_p = _fps_pallas(xyz)
    new_xyz = jnp.transpose(new_xyz_p, (0, 2, 1))
    sqr = _square_distance(new_xyz, xyz_t)
    _, knn_idx = jax.lax.top_k(-sqr, K)
    grouped_xyz = _group(xyz_t, knn_idx)
    grouped_norm = grouped_xyz - new_xyz[:, :, None, :]
    grouped_pts = _group(pts_t, knn_idx)
    feat = jnp.concatenate([grouped_norm, grouped_pts], axis=-1)
    x = jnp.transpose(feat, (0, 3, 1, 2))
    x = _lrelu(_bn2d(jnp.einsum('bcsk,cd->bdsk', x, W0), g0, b0))
    x = _lrelu(_bn2d(jnp.einsum('bcsk,cd->bdsk', x, W1), g1, b1))
    x = _lrelu(_bn2d(jnp.einsum('bcsk,cd->bdsk', x, W2), g2, b2))
    x = jnp.max(x, axis=-1)
    return (new_xyz_p, x, fps_idx)


# speedup vs baseline: 2.3966x; 1.3665x over previous
import jax
import jax.numpy as jnp
import numpy as np
from jax.experimental import pallas as pl
from jax.experimental.pallas import tpu as pltpu

B, N, S, K, D = 4, 8192, 1024, 32, 64
LEAKY = 0.1


def _fps_body(xyz_ref, idx_ref, nxyz_ref, dd_ref):
    x = xyz_ref[:, 0, :]
    y = xyz_ref[:, 1, :]
    z = xyz_ref[:, 2, :]
    iota = jax.lax.broadcasted_iota(jnp.int32, (B, N), 1)
    lane = jax.lax.broadcasted_iota(jnp.int32, (B, 128), 1)

    dd_ref[...] = jnp.full((B, N), 1e10, jnp.float32)

    def body(j, carry):
        cx, cy, cz, ai, ax, ay, az = carry
        dx = x - cx
        dy = y - cy
        dz = z - cz
        d = dx * dx + dy * dy + dz * dz
        dd = jnp.minimum(dd_ref[...], d)
        dd_ref[...] = dd
        m = jnp.max(dd, axis=1, keepdims=True)
        cand = jnp.where(dd == m, iota, N)
        nxt = jnp.min(cand, axis=1, keepdims=True)
        msk = iota == nxt
        ncx = jnp.sum(jnp.where(msk, x, 0.0), axis=1, keepdims=True)
        ncy = jnp.sum(jnp.where(msk, y, 0.0), axis=1, keepdims=True)
        ncz = jnp.sum(jnp.where(msk, z, 0.0), axis=1, keepdims=True)
        sel = lane == j
        ai = jnp.where(sel, nxt, ai)
        ax = jnp.where(sel, ncx, ax)
        ay = jnp.where(sel, ncy, ay)
        az = jnp.where(sel, ncz, az)
        return (ncx, ncy, ncz, ai, ax, ay, az)

    zi = jnp.zeros((B, 128), jnp.int32)
    zf = jnp.zeros((B, 128), jnp.float32)
    carry = (x[:, 0:1], y[:, 0:1], z[:, 0:1], zi,
             jnp.where(lane == 0, x[:, 0:1], zf),
             jnp.where(lane == 0, y[:, 0:1], zf),
             jnp.where(lane == 0, z[:, 0:1], zf))
    for chunk in range(S // 128):
        lo = 1 if chunk == 0 else 0
        carry = jax.lax.fori_loop(lo, 128, body, carry)
        cx, cy, cz, ai, ax, ay, az = carry
        sl = slice(chunk * 128, (chunk + 1) * 128)
        idx_ref[:, sl] = ai
        nxyz_ref[:, 0, sl] = ax
        nxyz_ref[:, 1, sl] = ay
        nxyz_ref[:, 2, sl] = az
        carry = (cx, cy, cz, zi, zf, zf, zf)


def _fps_pallas(xyz):
    return pl.pallas_call(
        _fps_body,
        out_shape=(
            jax.ShapeDtypeStruct((B, S), jnp.int32),
            jax.ShapeDtypeStruct((B, 3, S), jnp.float32),
        ),
        scratch_shapes=[pltpu.VMEM((B, N), jnp.float32)],
    )(xyz)


def _square_distance(src, dst):
    d = -2.0 * jnp.einsum('bnc,bmc->bnm', src, dst)
    d = d + jnp.sum(src ** 2, -1)[:, :, None]
    d = d + jnp.sum(dst ** 2, -1)[:, None, :]
    return d


def _lrelu(x):
    return jnp.where(x >= 0, x, LEAKY * x)


NROWS = B * S * K
TROWS = 512
NT = NROWS // TROWS
CNT = float(NROWS)


def _g_body(x3_ref, xp_ref, w0a_ref, w0b_ref, g_ref):
    x3 = x3_ref[...].reshape(TROWS, 3)
    xp = xp_ref[...].reshape(TROWS, D)
    g = jnp.dot(x3, w0a_ref[...], preferred_element_type=jnp.float32)
    g = g + jnp.dot(xp, w0b_ref[...], preferred_element_type=jnp.float32)
    g_ref[...] = g


def _g_pallas(xyz_t, pts_t, w0a, w0b):
    nt = N // TROWS
    return pl.pallas_call(
        _g_body,
        grid=(B, nt),
        in_specs=[
            pl.BlockSpec((1, TROWS, 3), lambda b, t: (b, t, 0)),
            pl.BlockSpec((1, TROWS, D), lambda b, t: (b, t, 0)),
            pl.BlockSpec((3, 64), lambda b, t: (0, 0)),
            pl.BlockSpec((D, 64), lambda b, t: (0, 0)),
        ],
        out_specs=pl.BlockSpec((TROWS, 64), lambda b, t: (b * nt + t, 0)),
        out_shape=jax.ShapeDtypeStruct((B * N, 64), jnp.float32),
    )(xyz_t, pts_t, w0a, w0b)


def _c_body(nx_ref, w0a_ref, c_ref):
    c_ref[...] = jnp.dot(nx_ref[...], w0a_ref[...],
                         preferred_element_type=jnp.float32)


def _c_pallas(nx_flat, w0a):
    nt = (B * S) // TROWS
    return pl.pallas_call(
        _c_body,
        grid=(nt,),
        in_specs=[
            pl.BlockSpec((TROWS, 3), lambda t: (t, 0)),
            pl.BlockSpec((3, 64), lambda t: (0, 0)),
        ],
        out_specs=pl.BlockSpec((TROWS, 64), lambda t: (t, 0)),
        out_shape=jax.ShapeDtypeStruct((B * S, 64), jnp.float32),
    )(nx_flat, w0a)


def _p1_body(y0g_ref, c_ref, st_ref, acc_ref):
    t = pl.program_id(0)

    @pl.when(t == 0)
    def _():
        acc_ref[...] = jnp.zeros((16, 64), jnp.float32)

    y0 = y0g_ref[...].reshape(16, K, 64) - c_ref[...].reshape(16, 1, 64)
    y0 = y0.reshape(TROWS, 64)
    ps = jnp.sum(y0.reshape(64, 8, 64), axis=0)
    pq = jnp.sum((y0 * y0).reshape(64, 8, 64), axis=0)
    acc_ref[0:8, :] = acc_ref[0:8, :] + ps
    acc_ref[8:16, :] = acc_ref[8:16, :] + pq

    @pl.when(t == NT - 1)
    def _():
        st_ref[...] = acc_ref[...]


def _p1_pallas(y0g, c):
    return pl.pallas_call(
        _p1_body,
        grid=(NT,),
        in_specs=[
            pl.BlockSpec((TROWS, 64), lambda t: (t, 0)),
            pl.BlockSpec((16, 64), lambda t: (t, 0)),
        ],
        out_specs=pl.BlockSpec((16, 64), lambda t: (0, 0)),
        out_shape=jax.ShapeDtypeStruct((16, 64), jnp.float32),
        scratch_shapes=[pltpu.VMEM((16, 64), jnp.float32)],
    )(y0g, c)


def _bn_coefs(st_ref, g_ref, b_ref, cnt):
    m = jnp.sum(st_ref[0:8, :], axis=0, keepdims=True) / cnt
    q = jnp.sum(st_ref[8:16, :], axis=0, keepdims=True) / cnt
    v = q - m * m
    s = g_ref[...] * jax.lax.rsqrt(v + 1e-5)
    c = b_ref[...] - m * s
    return s, c


def _p2_body(y0g_ref, c_ref, st0_ref, g0_ref, b0_ref, w1_ref,
             z1_ref, st_ref, acc_ref):
    t = pl.program_id(0)

    @pl.when(t == 0)
    def _():
        acc_ref[...] = jnp.zeros((16, 64), jnp.float32)

    s0, c0 = _bn_coefs(st0_ref, g0_ref, b0_ref, CNT)
    y0 = y0g_ref[...].reshape(16, K, 64) - c_ref[...].reshape(16, 1, 64)
    y0 = y0.reshape(TROWS, 64)
    a = _lrelu(y0 * s0 + c0)
    z1 = jnp.dot(a, w1_ref[...], preferred_element_type=jnp.float32)
    z1_ref[...] = z1
    acc_ref[0:8, :] = acc_ref[0:8, :] + jnp.sum(z1.reshape(64, 8, 64), axis=0)
    acc_ref[8:16, :] = acc_ref[8:16, :] + jnp.sum((z1 * z1).reshape(64, 8, 64), axis=0)

    @pl.when(t == NT - 1)
    def _():
        st_ref[...] = acc_ref[...]


def _p2_pallas(y0g, c, st0, g0, b0, w1):
    return pl.pallas_call(
        _p2_body,
        grid=(NT,),
        in_specs=[
            pl.BlockSpec((TROWS, 64), lambda t: (t, 0)),
            pl.BlockSpec((16, 64), lambda t: (t, 0)),
            pl.BlockSpec((16, 64), lambda t: (0, 0)),
            pl.BlockSpec((1, 64), lambda t: (0, 0)),
            pl.BlockSpec((1, 64), lambda t: (0, 0)),
            pl.BlockSpec((64, 64), lambda t: (0, 0)),
        ],
        out_specs=(
            pl.BlockSpec((TROWS, 64), lambda t: (t, 0)),
            pl.BlockSpec((16, 64), lambda t: (0, 0)),
        ),
        out_shape=(
            jax.ShapeDtypeStruct((NROWS, 64), jnp.float32),
            jax.ShapeDtypeStruct((16, 64), jnp.float32),
        ),
        scratch_shapes=[pltpu.VMEM((16, 64), jnp.float32)],
    )(y0g, c, st0, g0, b0, w1)


def _p3_body(z1_ref, st1_ref, g1_ref, b1_ref, w2_ref,
             mx_ref, mn_ref, st_ref, acc_ref):
    t = pl.program_id(0)

    @pl.when(t == 0)
    def _():
        acc_ref[...] = jnp.zeros((16, 128), jnp.float32)

    s1, c1 = _bn_coefs(st1_ref, g1_ref, b1_ref, CNT)
    a = _lrelu(z1_ref[...] * s1 + c1)
    z2 = jnp.dot(a, w2_ref[...], preferred_element_type=jnp.float32)
    acc_ref[0:8, :] = acc_ref[0:8, :] + jnp.sum(z2.reshape(64, 8, 128), axis=0)
    acc_ref[8:16, :] = acc_ref[8:16, :] + jnp.sum((z2 * z2).reshape(64, 8, 128), axis=0)
    z2r = z2.reshape(16, K, 128)
    mx_ref[...] = jnp.max(z2r, axis=1)
    mn_ref[...] = jnp.min(z2r, axis=1)

    @pl.when(t == NT - 1)
    def _():
        st_ref[...] = acc_ref[...]


def _p3_pallas(z1, st1, g1, b1, w2):
    return pl.pallas_call(
        _p3_body,
        grid=(NT,),
        in_specs=[
            pl.BlockSpec((TROWS, 64), lambda t: (t, 0)),
            pl.BlockSpec((16, 64), lambda t: (0, 0)),
            pl.BlockSpec((1, 64), lambda t: (0, 0)),
            pl.BlockSpec((1, 64), lambda t: (0, 0)),
            pl.BlockSpec((64, 128), lambda t: (0, 0)),
        ],
        out_specs=(
            pl.BlockSpec((16, 128), lambda t: (t, 0)),
            pl.BlockSpec((16, 128), lambda t: (t, 0)),
            pl.BlockSpec((16, 128), lambda t: (0, 0)),
        ),
        out_shape=(
            jax.ShapeDtypeStruct((B * S, 128), jnp.float32),
            jax.ShapeDtypeStruct((B * S, 128), jnp.float32),
            jax.ShapeDtypeStruct((16, 128), jnp.float32),
        ),
        scratch_shapes=[pltpu.VMEM((16, 128), jnp.float32)],
    )(z1, st1, g1, b1, w2)


def _p4_body(mx_ref, mn_ref, st2_ref, g2_ref, b2_ref, o_ref):
    s2, c2 = _bn_coefs(st2_ref, g2_ref, b2_ref, CNT)
    v = jnp.where(s2 >= 0, mx_ref[...] * s2, mn_ref[...] * s2) + c2
    o_ref[...] = _lrelu(v)


def _p4_pallas(mx, mn, st2, g2, b2):
    nt = (B * S) // TROWS
    return pl.pallas_call(
        _p4_body,
        grid=(nt,),
        in_specs=[
            pl.BlockSpec((TROWS, 128), lambda t: (t, 0)),
            pl.BlockSpec((TROWS, 128), lambda t: (t, 0)),
            pl.BlockSpec((16, 128), lambda t: (0, 0)),
            pl.BlockSpec((1, 128), lambda t: (0, 0)),
            pl.BlockSpec((1, 128), lambda t: (0, 0)),
        ],
        out_specs=pl.BlockSpec((TROWS, 128), lambda t: (t, 0)),
        out_shape=jax.ShapeDtypeStruct((B * S, 128), jnp.float32),
    )(mx, mn, st2, g2, b2)


def kernel(xyz, points, W0, W1, W2, g0, g1, g2, b0, b1, b2):
    xyz_t = jnp.transpose(xyz, (0, 2, 1))
    pts_t = jnp.transpose(points, (0, 2, 1))
    fps_idx, new_xyz_p = _fps_pallas(xyz)
    new_xyz = jnp.transpose(new_xyz_p, (0, 2, 1))
    sqr = _square_distance(new_xyz, xyz_t)
    _, knn_idx = jax.lax.top_k(-sqr, K)

    w0a, w0b = W0[:3], W0[3:]
    G = _g_pallas(xyz_t, pts_t, w0a, w0b)
    C = _c_pallas(new_xyz.reshape(B * S, 3), w0a)

    gidx = (knn_idx + (jnp.arange(B, dtype=jnp.int32) * N)[:, None, None]
            ).reshape(-1)
    y0g = jnp.take(G, gidx, axis=0)

    st0 = _p1_pallas(y0g, C)
    z1, st1 = _p2_pallas(y0g, C, st0, g0.reshape(1, 64), b0.reshape(1, 64), W1)
    mx, mn, st2 = _p3_pallas(z1, st1, g1.reshape(1, 64), b1.reshape(1, 64), W2)
    xo = _p4_pallas(mx, mn, st2, g2.reshape(1, 128), b2.reshape(1, 128))
    x = xo.reshape(B, S, 128).transpose(0, 2, 1)
    return (new_xyz_p, x, fps_idx)


# speedup vs baseline: 2.4814x; 1.0354x over previous
import functools

import jax
import jax.numpy as jnp
import numpy as np
from jax import lax
from jax.experimental import pallas as pl
from jax.experimental.pallas import tpu as pltpu
from jax.experimental.pallas import tpu_sc as plsc

B, N, S, K, D = 4, 8192, 1024, 32, 64
LEAKY = 0.1


def _fps_body(xyz_ref, idx_ref, nxyz_ref, dd_ref):
    x = xyz_ref[:, 0, :]
    y = xyz_ref[:, 1, :]
    z = xyz_ref[:, 2, :]
    iota = jax.lax.broadcasted_iota(jnp.int32, (B, N), 1)
    lane = jax.lax.broadcasted_iota(jnp.int32, (B, 128), 1)

    dd_ref[...] = jnp.full((B, N), 1e10, jnp.float32)

    def body(j, carry):
        cx, cy, cz, ai, ax, ay, az = carry
        dx = x - cx
        dy = y - cy
        dz = z - cz
        d = dx * dx + dy * dy + dz * dz
        dd = jnp.minimum(dd_ref[...], d)
        dd_ref[...] = dd
        m = jnp.max(dd, axis=1, keepdims=True)
        cand = jnp.where(dd == m, iota, N)
        nxt = jnp.min(cand, axis=1, keepdims=True)
        msk = iota == nxt
        ncx = jnp.sum(jnp.where(msk, x, 0.0), axis=1, keepdims=True)
        ncy = jnp.sum(jnp.where(msk, y, 0.0), axis=1, keepdims=True)
        ncz = jnp.sum(jnp.where(msk, z, 0.0), axis=1, keepdims=True)
        sel = lane == j
        ai = jnp.where(sel, nxt, ai)
        ax = jnp.where(sel, ncx, ax)
        ay = jnp.where(sel, ncy, ay)
        az = jnp.where(sel, ncz, az)
        return (ncx, ncy, ncz, ai, ax, ay, az)

    zi = jnp.zeros((B, 128), jnp.int32)
    zf = jnp.zeros((B, 128), jnp.float32)
    carry = (x[:, 0:1], y[:, 0:1], z[:, 0:1], zi,
             jnp.where(lane == 0, x[:, 0:1], zf),
             jnp.where(lane == 0, y[:, 0:1], zf),
             jnp.where(lane == 0, z[:, 0:1], zf))
    for chunk in range(S // 128):
        lo = 1 if chunk == 0 else 0
        carry = jax.lax.fori_loop(lo, 128, body, carry)
        cx, cy, cz, ai, ax, ay, az = carry
        sl = slice(chunk * 128, (chunk + 1) * 128)
        idx_ref[:, sl] = ai
        nxyz_ref[:, 0, sl] = ax
        nxyz_ref[:, 1, sl] = ay
        nxyz_ref[:, 2, sl] = az
        carry = (cx, cy, cz, zi, zf, zf, zf)


def _fps_pallas(xyz):
    return pl.pallas_call(
        _fps_body,
        out_shape=(
            jax.ShapeDtypeStruct((B, S), jnp.int32),
            jax.ShapeDtypeStruct((B, 3, S), jnp.float32),
        ),
        scratch_shapes=[pltpu.VMEM((B, N), jnp.float32)],
    )(xyz)


def _square_distance(src, dst):
    d = -2.0 * jnp.einsum('bnc,bmc->bnm', src, dst)
    d = d + jnp.sum(src ** 2, -1)[:, :, None]
    d = d + jnp.sum(dst ** 2, -1)[:, None, :]
    return d


def _lrelu(x):
    return jnp.where(x >= 0, x, LEAKY * x)


NROWS = B * S * K
TROWS = 512
NT = NROWS // TROWS
CNT = float(NROWS)


def _g_body(x3_ref, xp_ref, w0a_ref, w0b_ref, g_ref):
    x3 = x3_ref[...].reshape(TROWS, 3)
    xp = xp_ref[...].reshape(TROWS, D)
    g = jnp.dot(x3, w0a_ref[...], preferred_element_type=jnp.float32)
    g = g + jnp.dot(xp, w0b_ref[...], preferred_element_type=jnp.float32)
    g_ref[...] = g


def _g_pallas(xyz_t, pts_t, w0a, w0b):
    nt = N // TROWS
    return pl.pallas_call(
        _g_body,
        grid=(B, nt),
        in_specs=[
            pl.BlockSpec((1, TROWS, 3), lambda b, t: (b, t, 0)),
            pl.BlockSpec((1, TROWS, D), lambda b, t: (b, t, 0)),
            pl.BlockSpec((3, 128), lambda b, t: (0, 0)),
            pl.BlockSpec((D, 128), lambda b, t: (0, 0)),
        ],
        out_specs=pl.BlockSpec((TROWS, 128), lambda b, t: (b * nt + t, 0)),
        out_shape=jax.ShapeDtypeStruct((B * N, 128), jnp.float32),
    )(xyz_t, pts_t, w0a, w0b)


def _c_body(nx_ref, w0a_ref, c_ref):
    c_ref[...] = jnp.dot(nx_ref[...], w0a_ref[...],
                         preferred_element_type=jnp.float32)


def _c_pallas(nx_flat, w0a):
    nt = (B * S) // TROWS
    return pl.pallas_call(
        _c_body,
        grid=(nt,),
        in_specs=[
            pl.BlockSpec((TROWS, 3), lambda t: (t, 0)),
            pl.BlockSpec((3, 128), lambda t: (0, 0)),
        ],
        out_specs=pl.BlockSpec((TROWS, 128), lambda t: (t, 0)),
        out_shape=jax.ShapeDtypeStruct((B * S, 128), jnp.float32),
    )(nx_flat, w0a)


_NW = 32
_RPW = NROWS // _NW
_GCH = 128
_NCH = _RPW // _GCH


def _sc_gather_body(g_hbm, gidx_hbm, out_hbm, idx_v, rows_v, sem):
    wid = lax.axis_index("s") * 2 + lax.axis_index("c")

    def chunk(c, _):
        base = pl.multiple_of((wid * _NCH + c) * _GCH, _GCH)
        pltpu.sync_copy(gidx_hbm.at[pl.ds(base, _GCH)], idx_v)
        pltpu.async_copy(g_hbm.at[idx_v], rows_v, sem).wait()
        pltpu.sync_copy(rows_v, out_hbm.at[pl.ds(base, _GCH)])
        return 0

    lax.fori_loop(0, _NCH, chunk, 0)


def _sc_gather(G, gidx):
    mesh = plsc.VectorSubcoreMesh(core_axis_name="c", subcore_axis_name="s")
    fn = functools.partial(
        pl.kernel,
        out_type=jax.ShapeDtypeStruct((NROWS, 128), jnp.float32),
        mesh=mesh,
        scratch_types=[
            pltpu.VMEM((_GCH,), jnp.int32),
            pltpu.VMEM((_GCH, 128), jnp.float32),
            pltpu.SemaphoreType.DMA,
        ],
    )(_sc_gather_body)
    return fn(G, gidx)


def _p1_body(y0g_ref, c_ref, st_ref, acc_ref):
    t = pl.program_id(0)

    @pl.when(t == 0)
    def _():
        acc_ref[...] = jnp.zeros((16, 128), jnp.float32)

    y0 = y0g_ref[...].reshape(16, K, 128) - c_ref[...].reshape(16, 1, 128)
    y0 = y0.reshape(TROWS, 128)
    ps = jnp.sum(y0.reshape(64, 8, 128), axis=0)
    pq = jnp.sum((y0 * y0).reshape(64, 8, 128), axis=0)
    acc_ref[0:8, :] = acc_ref[0:8, :] + ps
    acc_ref[8:16, :] = acc_ref[8:16, :] + pq

    @pl.when(t == NT - 1)
    def _():
        st_ref[...] = acc_ref[...]


def _p1_pallas(y0g, c):
    return pl.pallas_call(
        _p1_body,
        grid=(NT,),
        in_specs=[
            pl.BlockSpec((TROWS, 128), lambda t: (t, 0)),
            pl.BlockSpec((16, 128), lambda t: (t, 0)),
        ],
        out_specs=pl.BlockSpec((16, 128), lambda t: (0, 0)),
        out_shape=jax.ShapeDtypeStruct((16, 128), jnp.float32),
        scratch_shapes=[pltpu.VMEM((16, 128), jnp.float32)],
    )(y0g, c)


def _bn_coefs(st_ref, g_ref, b_ref, cnt):
    m = jnp.sum(st_ref[0:8, :], axis=0, keepdims=True) / cnt
    q = jnp.sum(st_ref[8:16, :], axis=0, keepdims=True) / cnt
    v = q - m * m
    s = g_ref[...] * jax.lax.rsqrt(v + 1e-5)
    c = b_ref[...] - m * s
    return s, c


def _p2_body(y0g_ref, c_ref, st0_ref, g0_ref, b0_ref, w1_ref,
             z1_ref, st_ref, acc_ref):
    t = pl.program_id(0)

    @pl.when(t == 0)
    def _():
        acc_ref[...] = jnp.zeros((16, 64), jnp.float32)

    s0, c0 = _bn_coefs(st0_ref, g0_ref, b0_ref, CNT)
    y0 = y0g_ref[...].reshape(16, K, 128) - c_ref[...].reshape(16, 1, 128)
    y0 = y0.reshape(TROWS, 128)
    a = _lrelu(y0 * s0 + c0)
    z1 = jnp.dot(a, w1_ref[...], preferred_element_type=jnp.float32)
    z1_ref[...] = z1
    acc_ref[0:8, :] = acc_ref[0:8, :] + jnp.sum(z1.reshape(64, 8, 64), axis=0)
    acc_ref[8:16, :] = acc_ref[8:16, :] + jnp.sum((z1 * z1).reshape(64, 8, 64), axis=0)

    @pl.when(t == NT - 1)
    def _():
        st_ref[...] = acc_ref[...]


def _p2_pallas(y0g, c, st0, g0, b0, w1):
    return pl.pallas_call(
        _p2_body,
        grid=(NT,),
        in_specs=[
            pl.BlockSpec((TROWS, 128), lambda t: (t, 0)),
            pl.BlockSpec((16, 128), lambda t: (t, 0)),
            pl.BlockSpec((16, 128), lambda t: (0, 0)),
            pl.BlockSpec((1, 128), lambda t: (0, 0)),
            pl.BlockSpec((1, 128), lambda t: (0, 0)),
            pl.BlockSpec((128, 64), lambda t: (0, 0)),
        ],
        out_specs=(
            pl.BlockSpec((TROWS, 64), lambda t: (t, 0)),
            pl.BlockSpec((16, 64), lambda t: (0, 0)),
        ),
        out_shape=(
            jax.ShapeDtypeStruct((NROWS, 64), jnp.float32),
            jax.ShapeDtypeStruct((16, 64), jnp.float32),
        ),
        scratch_shapes=[pltpu.VMEM((16, 64), jnp.float32)],
    )(y0g, c, st0, g0, b0, w1)


def _p3_body(z1_ref, st1_ref, g1_ref, b1_ref, w2_ref,
             mx_ref, mn_ref, st_ref, acc_ref):
    t = pl.program_id(0)

    @pl.when(t == 0)
    def _():
        acc_ref[...] = jnp.zeros((16, 128), jnp.float32)

    s1, c1 = _bn_coefs(st1_ref, g1_ref, b1_ref, CNT)
    a = _lrelu(z1_ref[...] * s1 + c1)
    z2 = jnp.dot(a, w2_ref[...], preferred_element_type=jnp.float32)
    acc_ref[0:8, :] = acc_ref[0:8, :] + jnp.sum(z2.reshape(64, 8, 128), axis=0)
    acc_ref[8:16, :] = acc_ref[8:16, :] + jnp.sum((z2 * z2).reshape(64, 8, 128), axis=0)
    z2r = z2.reshape(16, K, 128)
    mx_ref[...] = jnp.max(z2r, axis=1)
    mn_ref[...] = jnp.min(z2r, axis=1)

    @pl.when(t == NT - 1)
    def _():
        st_ref[...] = acc_ref[...]


def _p3_pallas(z1, st1, g1, b1, w2):
    return pl.pallas_call(
        _p3_body,
        grid=(NT,),
        in_specs=[
            pl.BlockSpec((TROWS, 64), lambda t: (t, 0)),
            pl.BlockSpec((16, 64), lambda t: (0, 0)),
            pl.BlockSpec((1, 64), lambda t: (0, 0)),
            pl.BlockSpec((1, 64), lambda t: (0, 0)),
            pl.BlockSpec((64, 128), lambda t: (0, 0)),
        ],
        out_specs=(
            pl.BlockSpec((16, 128), lambda t: (t, 0)),
            pl.BlockSpec((16, 128), lambda t: (t, 0)),
            pl.BlockSpec((16, 128), lambda t: (0, 0)),
        ),
        out_shape=(
            jax.ShapeDtypeStruct((B * S, 128), jnp.float32),
            jax.ShapeDtypeStruct((B * S, 128), jnp.float32),
            jax.ShapeDtypeStruct((16, 128), jnp.float32),
        ),
        scratch_shapes=[pltpu.VMEM((16, 128), jnp.float32)],
    )(z1, st1, g1, b1, w2)


def _p4_body(mx_ref, mn_ref, st2_ref, g2_ref, b2_ref, o_ref):
    s2, c2 = _bn_coefs(st2_ref, g2_ref, b2_ref, CNT)
    v = jnp.where(s2 >= 0, mx_ref[...] * s2, mn_ref[...] * s2) + c2
    o_ref[...] = _lrelu(v)


def _p4_pallas(mx, mn, st2, g2, b2):
    nt = (B * S) // TROWS
    return pl.pallas_call(
        _p4_body,
        grid=(nt,),
        in_specs=[
            pl.BlockSpec((TROWS, 128), lambda t: (t, 0)),
            pl.BlockSpec((TROWS, 128), lambda t: (t, 0)),
            pl.BlockSpec((16, 128), lambda t: (0, 0)),
            pl.BlockSpec((1, 128), lambda t: (0, 0)),
            pl.BlockSpec((1, 128), lambda t: (0, 0)),
        ],
        out_specs=pl.BlockSpec((TROWS, 128), lambda t: (t, 0)),
        out_shape=jax.ShapeDtypeStruct((B * S, 128), jnp.float32),
    )(mx, mn, st2, g2, b2)


def kernel(xyz, points, W0, W1, W2, g0, g1, g2, b0, b1, b2):
    xyz_t = jnp.transpose(xyz, (0, 2, 1))
    pts_t = jnp.transpose(points, (0, 2, 1))
    fps_idx, new_xyz_p = _fps_pallas(xyz)
    new_xyz = jnp.transpose(new_xyz_p, (0, 2, 1))
    sqr = _square_distance(new_xyz, xyz_t)
    _, knn_idx = jax.lax.top_k(-sqr, K)

    w0a = jnp.pad(W0[:3], ((0, 0), (0, 64)))
    w0b = jnp.pad(W0[3:], ((0, 0), (0, 64)))
    w1p = jnp.pad(W1, ((0, 64), (0, 0)))
    g0p = jnp.pad(g0.reshape(1, 64), ((0, 0), (0, 64)))
    b0p = jnp.pad(b0.reshape(1, 64), ((0, 0), (0, 64)))
    G = _g_pallas(xyz_t, pts_t, w0a, w0b)
    C = _c_pallas(new_xyz.reshape(B * S, 3), w0a)

    gidx = (knn_idx + (jnp.arange(B, dtype=jnp.int32) * N)[:, None, None]
            ).reshape(-1)
    y0g = _sc_gather(G, gidx)

    st0 = _p1_pallas(y0g, C)
    z1, st1 = _p2_pallas(y0g, C, st0, g0p, b0p, w1p)
    mx, mn, st2 = _p3_pallas(z1, st1, g1.reshape(1, 64), b1.reshape(1, 64), W2)
    xo = _p4_pallas(mx, mn, st2, g2.reshape(1, 128), b2.reshape(1, 128))
    x = xo.reshape(B, S, 128).transpose(0, 2, 1)
    return (new_xyz_p, x, fps_idx)
